# baseline (speedup 1.0000x reference)
import jax
import jax.numpy as jnp
from jax.experimental import pallas as pl
from jax.experimental.pallas import tpu as pltpu

_SCALE = 80.0
_R2 = 40.0 * 40.0
_BIG = 1e10
_TN = 512


def _chamfer_kernel(p_ref, gt_ref, out_ref):
    N = p_ref.shape[1]
    M = gt_ref.shape[2]

    gx = gt_ref[0, 0:1, :] * _SCALE
    gy = gt_ref[0, 1:2, :] * _SCALE
    gz = gt_ref[0, 2:3, :] * _SCALE
    bb = gx * gx + gy * gy + gz * gz
    mg = bb < _R2
    gxr = gx.astype(jnp.bfloat16).astype(jnp.float32)
    gyr = gy.astype(jnp.bfloat16).astype(jnp.float32)
    gzr = gz.astype(jnp.bfloat16).astype(jnp.float32)

    def body(j, carry):
        cmin_u, cmin_m, rsum_u, rsum_m, cnt_p = carry
        p_blk = p_ref[0, pl.ds(j * _TN, _TN), :] * _SCALE
        px = p_blk[:, 0:1]
        py = p_blk[:, 1:2]
        pz = p_blk[:, 2:3]
        aa = px * px + py * py + pz * pz
        mp = aa < _R2
        pxr = px.astype(jnp.bfloat16).astype(jnp.float32)
        pyr = py.astype(jnp.bfloat16).astype(jnp.float32)
        pzr = pz.astype(jnp.bfloat16).astype(jnp.float32)
        ab = pxr * gxr + pyr * gyr + pzr * gzr
        d2 = jnp.maximum(aa + bb - 2.0 * ab, 0.0)

        rmin_u = jnp.min(d2, axis=1, keepdims=True)
        cmin_u = jnp.minimum(cmin_u, jnp.min(d2, axis=0, keepdims=True))

        rmin_m = jnp.min(jnp.where(mg, d2, _BIG), axis=1, keepdims=True)
        cmin_m = jnp.minimum(
            cmin_m, jnp.min(jnp.where(mp, d2, _BIG), axis=0, keepdims=True))

        rsum_u = rsum_u + jnp.sum(rmin_u)
        rsum_m = rsum_m + jnp.sum(jnp.where(mp, rmin_m, 0.0))
        cnt_p = cnt_p + jnp.sum(mp.astype(jnp.float32))
        return cmin_u, cmin_m, rsum_u, rsum_m, cnt_p

    init = (
        jnp.full((1, M), _BIG, jnp.float32),
        jnp.full((1, M), _BIG, jnp.float32),
        jnp.float32(0.0),
        jnp.float32(0.0),
        jnp.float32(0.0),
    )
    cmin_u, cmin_m, rsum_u, rsum_m, cnt_p = jax.lax.fori_loop(
        0, N // _TN, body, init)

    sum_c_u = jnp.sum(cmin_u)
    sum_c_m = jnp.sum(jnp.where(mg, cmin_m, 0.0))
    cnt_g = jnp.sum(mg.astype(jnp.float32))

    non_filtered = rsum_u / N + sum_c_u / M
    filtered = (rsum_m / jnp.maximum(cnt_p, 1.0)
                + sum_c_m / jnp.maximum(cnt_g, 1.0))
    loss = 0.7 * filtered + 0.3 * non_filtered
    out_ref[:, :, :] = jnp.broadcast_to(loss, (1, 1, 1))


def kernel(image_pred, image_gt):
    B, N, _ = image_pred.shape
    M = image_gt.shape[1]
    gt_t = jnp.swapaxes(image_gt, 1, 2)

    per_batch = pl.pallas_call(
        _chamfer_kernel,
        grid=(B,),
        in_specs=[
            pl.BlockSpec((1, N, 3), lambda b: (b, 0, 0)),
            pl.BlockSpec((1, 3, M), lambda b: (b, 0, 0)),
        ],
        out_specs=pl.BlockSpec((1, 1, 1), lambda b: (b, 0, 0)),
        out_shape=jax.ShapeDtypeStruct((B, 1, 1), jnp.float32),
        compiler_params=pltpu.CompilerParams(
            dimension_semantics=("parallel",)),
    )(image_pred, gt_t)
    return jnp.mean(per_batch)


# speedup vs baseline: 1.3764x; 1.3764x over previous
import jax
import jax.numpy as jnp
from jax.experimental import pallas as pl
from jax.experimental.pallas import tpu as pltpu

_SCALE = 80.0
_R2 = 40.0 * 40.0
_BIG = 1e10
_TN = 512


def _chamfer_kernel(p_ref, gt_ref, pr_ref, gr_ref, out_ref):
    N = p_ref.shape[1]
    M = gt_ref.shape[2]

    gx = gt_ref[0, 0:1, :] * _SCALE
    gy = gt_ref[0, 1:2, :] * _SCALE
    gz = gt_ref[0, 2:3, :] * _SCALE
    bb = gx * gx + gy * gy + gz * gz
    mg = bb < _R2
    g_r = gr_ref[0]

    def body(j, carry):
        cmin_u, cmin_m, rsum_u, rsum_m, cnt_p = carry
        p_blk = p_ref[0, pl.ds(j * _TN, _TN), :] * _SCALE
        px = p_blk[:, 0:1]
        py = p_blk[:, 1:2]
        pz = p_blk[:, 2:3]
        aa = px * px + py * py + pz * pz
        mp = aa < _R2

        p_r = pr_ref[0, pl.ds(j * _TN, _TN), :]
        ab = jax.lax.dot_general(
            p_r, g_r, (((1,), (0,)), ((), ())),
            preferred_element_type=jnp.float32)
        d2 = jnp.maximum(aa + bb - 2.0 * ab, 0.0)

        rmin_u = jnp.min(d2, axis=1, keepdims=True)
        cmin_u = jnp.minimum(cmin_u, jnp.min(d2, axis=0, keepdims=True))

        rmin_m = jnp.min(jnp.where(mg, d2, _BIG), axis=1, keepdims=True)
        cmin_m = jnp.minimum(
            cmin_m, jnp.min(jnp.where(mp, d2, _BIG), axis=0, keepdims=True))

        rsum_u = rsum_u + jnp.sum(rmin_u)
        rsum_m = rsum_m + jnp.sum(jnp.where(mp, rmin_m, 0.0))
        cnt_p = cnt_p + jnp.sum(mp.astype(jnp.float32))
        return cmin_u, cmin_m, rsum_u, rsum_m, cnt_p

    init = (
        jnp.full((1, M), _BIG, jnp.float32),
        jnp.full((1, M), _BIG, jnp.float32),
        jnp.float32(0.0),
        jnp.float32(0.0),
        jnp.float32(0.0),
    )
    cmin_u, cmin_m, rsum_u, rsum_m, cnt_p = jax.lax.fori_loop(
        0, N // _TN, body, init)

    sum_c_u = jnp.sum(cmin_u)
    sum_c_m = jnp.sum(jnp.where(mg, cmin_m, 0.0))
    cnt_g = jnp.sum(mg.astype(jnp.float32))

    non_filtered = rsum_u / N + sum_c_u / M
    filtered = (rsum_m / jnp.maximum(cnt_p, 1.0)
                + sum_c_m / jnp.maximum(cnt_g, 1.0))
    loss = 0.7 * filtered + 0.3 * non_filtered
    out_ref[:, :, :] = jnp.broadcast_to(loss, (1, 1, 1))


def kernel(image_pred, image_gt):
    B, N, _ = image_pred.shape
    M = image_gt.shape[1]
    gt_t = jnp.swapaxes(image_gt, 1, 2)

    p_r = (image_pred * _SCALE).astype(jnp.bfloat16)
    g_r = (image_gt * _SCALE).astype(jnp.bfloat16)
    p_r = jnp.pad(p_r, ((0, 0), (0, 0), (0, 5)))
    g_rt = jnp.pad(jnp.swapaxes(g_r, 1, 2),
                   ((0, 0), (0, 5), (0, 0)))

    per_batch = pl.pallas_call(
        _chamfer_kernel,
        grid=(B,),
        in_specs=[
            pl.BlockSpec((1, N, 3), lambda b: (b, 0, 0)),
            pl.BlockSpec((1, 3, M), lambda b: (b, 0, 0)),
            pl.BlockSpec((1, N, 8), lambda b: (b, 0, 0)),
            pl.BlockSpec((1, 8, M), lambda b: (b, 0, 0)),
        ],
        out_specs=pl.BlockSpec((1, 1, 1), lambda b: (b, 0, 0)),
        out_shape=jax.ShapeDtypeStruct((B, 1, 1), jnp.float32),
        compiler_params=pltpu.CompilerParams(
            dimension_semantics=("parallel",)),
    )(image_pred, gt_t, p_r, g_rt)
    return jnp.mean(per_batch)
